# baseline (speedup 1.0000x reference)
import functools

import jax
import jax.numpy as jnp
from jax import lax
from jax.experimental import pallas as pl
from jax.experimental.pallas import tpu as pltpu
from jax.experimental.pallas import tpu_sc as plsc

_NC = 2
_NS = 16
_NW = _NC * _NS


def _sc_gather_rows(scales, shifts, labels):
    B = labels.shape[0]
    V, D = scales.shape
    b_per_w = B // _NW
    mesh = plsc.VectorSubcoreMesh(core_axis_name="c", subcore_axis_name="s")

    @functools.partial(
        pl.kernel,
        mesh=mesh,
        out_type=(
            jax.ShapeDtypeStruct((B, D), scales.dtype),
            jax.ShapeDtypeStruct((B, D), shifts.dtype),
        ),
        scratch_types=[
            pltpu.VMEM((b_per_w,), jnp.int32),
            pltpu.VMEM((b_per_w, D), scales.dtype),
            pltpu.VMEM((b_per_w, D), shifts.dtype),
            pltpu.SemaphoreType.DMA,
            pltpu.SemaphoreType.DMA,
            pltpu.SemaphoreType.DMA,
            pltpu.SemaphoreType.DMA,
        ],
    )
    def gather_kernel(scales_hbm, shifts_hbm, idx_hbm, sc_out, sh_out,
                      idx_v, srows_v, hrows_v, sem_a, sem_b, sem_c, sem_d):
        wid = lax.axis_index("s") * _NC + lax.axis_index("c")
        base = wid * b_per_w
        pltpu.sync_copy(idx_hbm.at[pl.ds(base, b_per_w)], idx_v)
        ca = pltpu.async_copy(scales_hbm.at[idx_v], srows_v, sem_a)
        cb = pltpu.async_copy(shifts_hbm.at[idx_v], hrows_v, sem_b)
        ca.wait()
        cc = pltpu.async_copy(srows_v, sc_out.at[pl.ds(base, b_per_w)], sem_c)
        cb.wait()
        cd = pltpu.async_copy(hrows_v, sh_out.at[pl.ds(base, b_per_w)], sem_d)
        cc.wait()
        cd.wait()

    return gather_kernel(scales, shifts, labels)


def _tc_affine(x, scale_rows, shift_rows):
    S, B, W = x.shape
    blk_s = 2

    def body(x_ref, sc_ref, sh_ref, o_ref):
        o_ref[...] = x_ref[...] * sc_ref[...][None, :, :] + sh_ref[...][None, :, :]

    return pl.pallas_call(
        body,
        grid=(S // blk_s,),
        in_specs=[
            pl.BlockSpec((blk_s, B, W), lambda i: (i, 0, 0)),
            pl.BlockSpec((B, W), lambda i: (0, 0)),
            pl.BlockSpec((B, W), lambda i: (0, 0)),
        ],
        out_specs=pl.BlockSpec((blk_s, B, W), lambda i: (i, 0, 0)),
        out_shape=jax.ShapeDtypeStruct((S, B, W), x.dtype),
    )(x, scale_rows, shift_rows)


def kernel(x, task_labels, num_samples, scales, shifts):
    del num_samples
    labels = task_labels.astype(jnp.int32)
    scale_rows, shift_rows = _sc_gather_rows(scales, shifts, labels)
    return _tc_affine(x, scale_rows, shift_rows)


# speedup vs baseline: 1.0486x; 1.0486x over previous
import functools

import jax
import jax.numpy as jnp
from jax import lax
from jax.experimental import pallas as pl
from jax.experimental.pallas import tpu as pltpu
from jax.experimental.pallas import tpu_sc as plsc

_NC = 2
_NS = 16
_NW = _NC * _NS


def _sc_gather_rows(scales, shifts, labels):
    B = labels.shape[0]
    V, D = scales.shape
    b_per_w = B // _NW
    mesh = plsc.VectorSubcoreMesh(core_axis_name="c", subcore_axis_name="s")

    @functools.partial(
        pl.kernel,
        mesh=mesh,
        out_type=jax.ShapeDtypeStruct((2, B, D), scales.dtype),
        scratch_types=[
            pltpu.VMEM((b_per_w,), jnp.int32),
            pltpu.VMEM((b_per_w, D), scales.dtype),
            pltpu.VMEM((b_per_w, D), shifts.dtype),
            pltpu.SemaphoreType.DMA,
            pltpu.SemaphoreType.DMA,
            pltpu.SemaphoreType.DMA,
            pltpu.SemaphoreType.DMA,
        ],
    )
    def gather_kernel(scales_hbm, shifts_hbm, idx_hbm, rows_out,
                      idx_v, srows_v, hrows_v, sem_a, sem_b, sem_c, sem_d):
        wid = lax.axis_index("s") * _NC + lax.axis_index("c")
        base = wid * b_per_w
        pltpu.sync_copy(idx_hbm.at[pl.ds(base, b_per_w)], idx_v)
        ca = pltpu.async_copy(scales_hbm.at[idx_v], srows_v, sem_a)
        cb = pltpu.async_copy(shifts_hbm.at[idx_v], hrows_v, sem_b)
        ca.wait()
        cc = pltpu.async_copy(srows_v, rows_out.at[0, pl.ds(base, b_per_w)], sem_c)
        cb.wait()
        cd = pltpu.async_copy(hrows_v, rows_out.at[1, pl.ds(base, b_per_w)], sem_d)
        cc.wait()
        cd.wait()

    return gather_kernel(scales, shifts, labels)


def _tc_affine(x, rows):
    S, B, W = x.shape
    blk_b = 2048

    def body(x_ref, r_ref, o_ref):
        o_ref[...] = (x_ref[...] * r_ref[0][None, :, :]
                      + r_ref[1][None, :, :])

    return pl.pallas_call(
        body,
        grid=(B // blk_b,),
        in_specs=[
            pl.BlockSpec((S, blk_b, W), lambda i: (0, i, 0)),
            pl.BlockSpec((2, blk_b, W), lambda i: (0, i, 0)),
        ],
        out_specs=pl.BlockSpec((S, blk_b, W), lambda i: (0, i, 0)),
        out_shape=jax.ShapeDtypeStruct((S, B, W), x.dtype),
    )(x, rows)


def kernel(x, task_labels, num_samples, scales, shifts):
    del num_samples
    labels = task_labels.astype(jnp.int32)
    rows = _sc_gather_rows(scales, shifts, labels)
    return _tc_affine(x, rows)
